# baseline (speedup 1.0000x reference)
import functools

import jax
import jax.numpy as jnp
import numpy as np
from jax.experimental import pallas as pl
from jax.experimental.pallas import tpu as pltpu

_KPAD = 72
_EPS_BITS = int(np.float32(1e-12).view(np.int32))


def _medr_kernel(q_ref, et_ref, ew_ref, dists_ref, ranks_ref, ranksum_ref,
                 aug_ref):
    i = pl.program_id(0)
    bq = q_ref.shape[0]
    n = et_ref.shape[1]

    @pl.when(i == 0)
    def _build_aug():
        et = et_ref[...]
        aug_ref[0:64, :] = et
        aug_ref[64:65, :] = jnp.sum(et * et, axis=0, keepdims=True)
        aug_ref[65:66, :] = jnp.ones((1, n), jnp.float32)
        aug_ref[66:_KPAD, :] = jnp.zeros((_KPAD - 66, n), jnp.float32)

    q = q_ref[...]
    qn = jnp.sum(q * q, axis=1, keepdims=True)
    aug_q = jnp.concatenate(
        [-2.0 * q, jnp.ones((bq, 1), jnp.float32), qn,
         jnp.zeros((bq, _KPAD - 66), jnp.float32)], axis=1)

    ew_t = ew_ref[...].T
    aug_w = jnp.concatenate(
        [ew_t, jnp.sum(ew_t * ew_t, axis=0, keepdims=True),
         jnp.ones((1, 3 * bq), jnp.float32),
         jnp.zeros((_KPAD - 66, 3 * bq), jnp.float32)], axis=0)
    sq_w = jax.lax.dot_general(
        aug_q, aug_w,
        dimension_numbers=(((1,), (0,)), ((), ())),
        preferred_element_type=jnp.float32,
    )
    sqc_w = jnp.maximum(jax.lax.bitcast_convert_type(sq_w, jnp.int32),
                        _EPS_BITS)
    row = jax.lax.broadcasted_iota(jnp.int32, (bq, 1), 0) + i * bq
    p = row * 3 + 1
    col_w = jax.lax.broadcasted_iota(jnp.int32, (bq, 3 * bq), 1) + 3 * bq * i
    sqc_pos = jnp.sum(jnp.where(col_w == p, sqc_w, 0), axis=1, keepdims=True)

    sq = jax.lax.dot_general(
        aug_q, aug_ref[...],
        dimension_numbers=(((1,), (0,)), ((), ())),
        preferred_element_type=jnp.float32,
    )
    sqc_i = jnp.maximum(jax.lax.bitcast_convert_type(sq, jnp.int32), _EPS_BITS)
    sqc = jax.lax.bitcast_convert_type(sqc_i, jnp.float32)
    dists_ref[...] = sqc * jax.lax.rsqrt(sqc)

    rank = jnp.sum(jnp.where(sqc_i < sqc_pos, 1, 0), axis=1, keepdims=True) - 1
    ranks_ref[...] = rank

    @pl.when(i == 0)
    def _init_sum():
        ranksum_ref[...] = jnp.zeros_like(ranksum_ref)
    ranksum_ref[...] += jnp.sum(rank.astype(jnp.float32), keepdims=True)


@functools.partial(jax.jit, static_argnames=())
def _run(emb):
    n, dim = emb.shape
    q_count = n // 3
    queries = emb[0::3]
    emb_t = emb.T

    bq = 200
    grid = (q_count // bq,)

    dists, ranks, ranksum = pl.pallas_call(
        _medr_kernel,
        grid=grid,
        in_specs=[
            pl.BlockSpec((bq, dim), lambda i: (i, 0)),
            pl.BlockSpec((dim, n), lambda i: (0, 0)),
            pl.BlockSpec((3 * bq, dim), lambda i: (i, 0)),
        ],
        out_specs=[
            pl.BlockSpec((bq, n), lambda i: (i, 0)),
            pl.BlockSpec((bq, 1), lambda i: (i, 0)),
            pl.BlockSpec((1, 1), lambda i: (0, 0)),
        ],
        out_shape=[
            jax.ShapeDtypeStruct((q_count, n), jnp.float32),
            jax.ShapeDtypeStruct((q_count, 1), jnp.int32),
            jax.ShapeDtypeStruct((1, 1), jnp.float32),
        ],
        scratch_shapes=[pltpu.VMEM((_KPAD, n), jnp.float32)],
    )(queries, emb_t, emb)

    positive_ranks = ranks[:, 0]
    medr = ranksum[0, 0] / q_count
    return dists, positive_ranks, medr


def kernel(criterionOutput, networkOutput, batch):
    return _run(networkOutput)


# speedup vs baseline: 1.0654x; 1.0654x over previous
import functools

import jax
import jax.numpy as jnp
import numpy as np
from jax.experimental import pallas as pl
from jax.experimental.pallas import tpu as pltpu

_KPAD = 72
_EPS_BITS = int(np.float32(1e-12).view(np.int32))


def _medr_kernel(e_ref, ew_ref, dists_ref, ranks_ref, ranksum_ref, aug_ref):
    i = pl.program_id(0)
    bq = ew_ref.shape[0] // 3
    n = e_ref.shape[0]

    @pl.when(i == 0)
    def _build_aug():
        et = e_ref[...].T
        aug_ref[0:64, :] = et
        aug_ref[64:65, :] = jnp.sum(et * et, axis=0, keepdims=True)
        aug_ref[65:66, :] = jnp.ones((1, n), jnp.float32)
        aug_ref[66:_KPAD, :] = jnp.zeros((_KPAD - 66, n), jnp.float32)

    dim = e_ref.shape[1]
    q = ew_ref[...].reshape(bq, 3, dim)[:, 0, :]
    qn = jnp.sum(q * q, axis=1, keepdims=True)
    aug_q = jnp.concatenate(
        [-2.0 * q, jnp.ones((bq, 1), jnp.float32), qn,
         jnp.zeros((bq, _KPAD - 66), jnp.float32)], axis=1)

    ew_t = ew_ref[...].T
    aug_w = jnp.concatenate(
        [ew_t, jnp.sum(ew_t * ew_t, axis=0, keepdims=True),
         jnp.ones((1, 3 * bq), jnp.float32),
         jnp.zeros((_KPAD - 66, 3 * bq), jnp.float32)], axis=0)
    sq_w = jax.lax.dot_general(
        aug_q, aug_w,
        dimension_numbers=(((1,), (0,)), ((), ())),
        preferred_element_type=jnp.float32,
    )
    sqc_w = jnp.maximum(jax.lax.bitcast_convert_type(sq_w, jnp.int32),
                        _EPS_BITS)
    row = jax.lax.broadcasted_iota(jnp.int32, (bq, 1), 0) + i * bq
    p = row * 3 + 1
    col_w = jax.lax.broadcasted_iota(jnp.int32, (bq, 3 * bq), 1) + 3 * bq * i
    sqc_pos = jnp.sum(jnp.where(col_w == p, sqc_w, 0), axis=1, keepdims=True)

    sq = jax.lax.dot_general(
        aug_q, aug_ref[...],
        dimension_numbers=(((1,), (0,)), ((), ())),
        preferred_element_type=jnp.float32,
    )
    sqc_i = jnp.maximum(jax.lax.bitcast_convert_type(sq, jnp.int32), _EPS_BITS)
    sqc = jax.lax.bitcast_convert_type(sqc_i, jnp.float32)
    dists_ref[...] = sqc * jax.lax.rsqrt(sqc)

    rank = jnp.sum(jnp.where(sqc_i < sqc_pos, 1, 0), axis=1, keepdims=True) - 1
    ranks_ref[...] = rank

    @pl.when(i == 0)
    def _init_sum():
        ranksum_ref[...] = jnp.zeros_like(ranksum_ref)
    ranksum_ref[...] += jnp.sum(rank.astype(jnp.float32), keepdims=True)


@functools.partial(jax.jit, static_argnames=())
def _run(emb):
    n, dim = emb.shape
    q_count = n // 3

    bq = 200
    grid = (q_count // bq,)

    dists, ranks, ranksum = pl.pallas_call(
        _medr_kernel,
        grid=grid,
        in_specs=[
            pl.BlockSpec((n, dim), lambda i: (0, 0)),
            pl.BlockSpec((3 * bq, dim), lambda i: (i, 0)),
        ],
        out_specs=[
            pl.BlockSpec((bq, n), lambda i: (i, 0)),
            pl.BlockSpec((bq, 1), lambda i: (i, 0)),
            pl.BlockSpec((1, 1), lambda i: (0, 0)),
        ],
        out_shape=[
            jax.ShapeDtypeStruct((q_count, n), jnp.float32),
            jax.ShapeDtypeStruct((q_count, 1), jnp.int32),
            jax.ShapeDtypeStruct((1, 1), jnp.float32),
        ],
        scratch_shapes=[pltpu.VMEM((_KPAD, n), jnp.float32)],
    )(emb, emb)

    positive_ranks = ranks[:, 0]
    medr = ranksum[0, 0] / q_count
    return dists, positive_ranks, medr


def kernel(criterionOutput, networkOutput, batch):
    return _run(networkOutput)
